# baseline (speedup 1.0000x reference)
import functools

import jax
import jax.numpy as jnp
from jax import lax
from jax.experimental import pallas as pl
from jax.experimental.pallas import tpu as pltpu
from jax.experimental.pallas import tpu_sc as plsc

E = 8
D = 2048
DI = 4096
T = 4096
K = 2
S = T * K

TM = 256
NP = S + E * TM
NT = NP // TM
NB = 512
NN = DI // NB

D2 = D // 2

NC, NS = 2, 16
NW = NC * NS

_SC_MESH = dict(core_axis_name="c", subcore_axis_name="s",
                num_cores=NC, num_subcores=NS)


def _routing(top_k_index, top_k_weights):
    expert = top_k_index.reshape(-1).astype(jnp.int32)
    token = jnp.arange(S, dtype=jnp.int32) // K
    order = jnp.argsort(expert, stable=True)
    sorted_expert = expert[order]
    counts = jnp.bincount(expert, length=E).astype(jnp.int32)
    group_off = jnp.concatenate(
        [jnp.zeros(1, jnp.int32), jnp.cumsum(counts)]).astype(jnp.int32)
    padded = ((counts + TM - 1) // TM) * TM
    padded_off = jnp.concatenate(
        [jnp.zeros(1, jnp.int32), jnp.cumsum(padded)]).astype(jnp.int32)
    pos = (padded_off[sorted_expert]
           + jnp.arange(S, dtype=jnp.int32) - group_off[sorted_expert])
    slot_token = jnp.zeros(NP, jnp.int32).at[pos].set(token[order])
    slot_weight = jnp.zeros(NP, jnp.float32).at[pos].set(
        top_k_weights.reshape(-1)[order])
    inv_pos = jnp.zeros(S, jnp.int32).at[order].set(pos).reshape(T, K)
    tile_expert = jnp.searchsorted(
        padded_off, jnp.arange(NT, dtype=jnp.int32) * TM,
        side="right").astype(jnp.int32) - 1
    tile_expert = jnp.clip(tile_expert, 0, E - 1)
    return slot_token, slot_weight, inv_pos, tile_expert


_G_CH = 32
_G_ROWS = NP // NW
_G_NCH = _G_ROWS // _G_CH


def _gather_body(x_hbm, idx_hbm, out_hbm, idx_v, buf0, buf1, sem0, sem1):
    wid = lax.axis_index("s") * NC + lax.axis_index("c")
    base = wid * _G_ROWS
    pltpu.sync_copy(idx_hbm.at[pl.ds(base, _G_ROWS)], idx_v)

    def start(j, buf, sem):
        pltpu.make_async_copy(
            x_hbm.at[idx_v.at[pl.ds(j * _G_CH, _G_CH)]], buf, sem).start()

    def wait(buf, sem):
        pltpu.make_async_copy(x_hbm.at[idx_v.at[pl.ds(0, _G_CH)]], buf,
                              sem).wait()

    start(0, buf0, sem0)

    def chunk(k, carry):
        j0 = 2 * k
        wait(buf0, sem0)
        start(j0 + 1, buf1, sem1)
        pltpu.sync_copy(buf0, out_hbm.at[pl.ds(base + j0 * _G_CH, _G_CH)])

        j1 = j0 + 1
        wait(buf1, sem1)

        @pl.when(j1 + 1 < _G_NCH)
        def _():
            start(j1 + 1, buf0, sem0)

        pltpu.sync_copy(buf1, out_hbm.at[pl.ds(base + j1 * _G_CH, _G_CH)])
        return carry

    lax.fori_loop(0, _G_NCH // 2, chunk, 0)


def _sc_gather(x2, slot_token):
    return pl.kernel(
        _gather_body,
        out_type=jax.ShapeDtypeStruct((NP, D2), jnp.int32),
        mesh=plsc.VectorSubcoreMesh(**_SC_MESH),
        scratch_types=[
            pltpu.VMEM((_G_ROWS,), jnp.int32),
            pltpu.VMEM((_G_CH, D2), jnp.int32),
            pltpu.VMEM((_G_CH, D2), jnp.int32),
            pltpu.SemaphoreType.DMA,
            pltpu.SemaphoreType.DMA,
        ],
    )(x2, slot_token)


def _ffn_body(te_ref, x_ref, g_ref, u_ref, d_ref, w_ref, a_ref, out_ref):
    n = pl.program_id(0)
    xb = x_ref[...]
    gw = g_ref[0].astype(jnp.bfloat16)
    uw = u_ref[0].astype(jnp.bfloat16)
    dw = d_ref[0].astype(jnp.bfloat16)
    g = jnp.dot(xb, gw, preferred_element_type=jnp.float32)
    u = jnp.dot(xb, uw, preferred_element_type=jnp.float32)
    h = jax.nn.gelu(g, approximate=True) * u
    p = jnp.dot(h.astype(jnp.bfloat16), dw,
                preferred_element_type=jnp.float32)
    p = p * w_ref[0, 0, :][:, None]

    @pl.when(n == 0)
    def _():
        out_ref[...] = p

    @pl.when(n != 0)
    def _():
        out_ref[...] = a_ref[...] + p


def _tc_ffn(x_g, gate_up_proj, down_proj, slot_weight, tile_expert):
    w3 = slot_weight.reshape(NT, 1, TM)
    acc = jnp.zeros((NP, D), jnp.float32)
    grid_spec = pltpu.PrefetchScalarGridSpec(
        num_scalar_prefetch=1,
        grid=(NN, NT),
        in_specs=[
            pl.BlockSpec((TM, D), lambda n, i, te: (i, 0)),
            pl.BlockSpec((1, D, NB), lambda n, i, te: (te[i], 0, n)),
            pl.BlockSpec((1, D, NB), lambda n, i, te: (te[i], 0, NN + n)),
            pl.BlockSpec((1, NB, D), lambda n, i, te: (te[i], n, 0)),
            pl.BlockSpec((1, 1, TM), lambda n, i, te: (i, 0, 0)),
            pl.BlockSpec((TM, D), lambda n, i, te: (i, 0)),
        ],
        out_specs=pl.BlockSpec((TM, D), lambda n, i, te: (i, 0)),
    )
    return pl.pallas_call(
        _ffn_body,
        grid_spec=grid_spec,
        out_shape=jax.ShapeDtypeStruct((NP, D), jnp.float32),
        input_output_aliases={6: 0},
        compiler_params=pltpu.CompilerParams(
            dimension_semantics=("arbitrary", "arbitrary")),
    )(tile_expert, x_g, gate_up_proj, gate_up_proj, down_proj, w3, acc)


_C_CH = 16
_C_TOK = T // NW
_VR = D // 16


def _combine_body(hg_hbm, p0_hbm, p1_hbm, out_hbm,
                  i0_v, i1_v, r0_v, r1_v, s0, s1):
    wid = lax.axis_index("s") * NC + lax.axis_index("c")
    base = wid * _C_TOK

    def chunk(i, carry):
        off = base + i * _C_CH
        pltpu.sync_copy(p0_hbm.at[pl.ds(off, _C_CH)], i0_v)
        pltpu.sync_copy(p1_hbm.at[pl.ds(off, _C_CH)], i1_v)
        c0 = pltpu.async_copy(hg_hbm.at[i0_v], r0_v, s0)
        c1 = pltpu.async_copy(hg_hbm.at[i1_v], r1_v, s1)
        c0.wait()
        c1.wait()

        def row(r, carry2):
            def vec(j, carry3):
                sl = pl.ds(j * 16, 16)
                r0_v[r, sl] = r0_v[r, sl] + r1_v[r, sl]
                return carry3
            return lax.fori_loop(0, _VR, vec, carry2, unroll=8)

        lax.fori_loop(0, _C_CH, row, 0)
        pltpu.sync_copy(r0_v, out_hbm.at[pl.ds(off, _C_CH)])
        return carry

    lax.fori_loop(0, _C_TOK // _C_CH, chunk, 0)


def _sc_combine(h_g, inv_pos):
    p0 = inv_pos[:, 0]
    p1 = inv_pos[:, 1]
    return pl.kernel(
        _combine_body,
        out_type=jax.ShapeDtypeStruct((T, D), jnp.float32),
        mesh=plsc.VectorSubcoreMesh(**_SC_MESH),
        scratch_types=[
            pltpu.VMEM((_C_CH,), jnp.int32),
            pltpu.VMEM((_C_CH,), jnp.int32),
            pltpu.VMEM((_C_CH, D), jnp.float32),
            pltpu.VMEM((_C_CH, D), jnp.float32),
            pltpu.SemaphoreType.DMA,
            pltpu.SemaphoreType.DMA,
        ],
    )(h_g, p0, p1)


def kernel(x, top_k_index, top_k_weights, gate_up_proj, down_proj):
    slot_token, slot_weight, inv_pos, tile_expert = _routing(
        top_k_index, top_k_weights)
    x2 = lax.bitcast_convert_type(
        x.astype(jnp.bfloat16).reshape(T, D2, 2), jnp.int32)
    xg2 = _sc_gather(x2, slot_token)
    x_g = lax.bitcast_convert_type(xg2, jnp.bfloat16).reshape(NP, D)
    h_g = _tc_ffn(x_g, gate_up_proj, down_proj, slot_weight, tile_expert)
    return _sc_combine(h_g, inv_pos)


# speedup vs baseline: 1.1391x; 1.1391x over previous
import functools

import jax
import jax.numpy as jnp
from jax import lax
from jax.experimental import pallas as pl
from jax.experimental.pallas import tpu as pltpu
from jax.experimental.pallas import tpu_sc as plsc

E = 8
D = 2048
DI = 4096
T = 4096
K = 2
S = T * K

TM = 256
NP = S + E * TM
NT = NP // TM
NB = 1024
NN = DI // NB

D2 = D // 2

NC, NS = 2, 16
NW = NC * NS

_SC_MESH = dict(core_axis_name="c", subcore_axis_name="s",
                num_cores=NC, num_subcores=NS)


def _routing(top_k_index, top_k_weights):
    expert = top_k_index.reshape(-1).astype(jnp.int32)
    token = jnp.arange(S, dtype=jnp.int32) // K
    order = jnp.argsort(expert, stable=True)
    sorted_expert = expert[order]
    counts = jnp.bincount(expert, length=E).astype(jnp.int32)
    group_off = jnp.concatenate(
        [jnp.zeros(1, jnp.int32), jnp.cumsum(counts)]).astype(jnp.int32)
    padded = ((counts + TM - 1) // TM) * TM
    padded_off = jnp.concatenate(
        [jnp.zeros(1, jnp.int32), jnp.cumsum(padded)]).astype(jnp.int32)
    pos = (padded_off[sorted_expert]
           + jnp.arange(S, dtype=jnp.int32) - group_off[sorted_expert])
    slot_token = jnp.zeros(NP, jnp.int32).at[pos].set(token[order])
    slot_weight = jnp.zeros(NP, jnp.float32).at[pos].set(
        top_k_weights.reshape(-1)[order])
    inv_pos = jnp.zeros(S, jnp.int32).at[order].set(pos).reshape(T, K)
    tile_expert = jnp.searchsorted(
        padded_off, jnp.arange(NT, dtype=jnp.int32) * TM,
        side="right").astype(jnp.int32) - 1
    tile_expert = jnp.clip(tile_expert, 0, E - 1)
    return slot_token, slot_weight, inv_pos, tile_expert


_G_CH = 32
_G_ROWS = NP // NW
_G_NCH = _G_ROWS // _G_CH


def _gather_body(x_hbm, idx_hbm, out_hbm, idx_v, buf0, buf1, sem0, sem1):
    wid = lax.axis_index("s") * NC + lax.axis_index("c")
    base = wid * _G_ROWS
    pltpu.sync_copy(idx_hbm.at[pl.ds(base, _G_ROWS)], idx_v)

    def start(j, buf, sem):
        pltpu.make_async_copy(
            x_hbm.at[idx_v.at[pl.ds(j * _G_CH, _G_CH)]], buf, sem).start()

    def wait(buf, sem):
        pltpu.make_async_copy(x_hbm.at[idx_v.at[pl.ds(0, _G_CH)]], buf,
                              sem).wait()

    start(0, buf0, sem0)

    def chunk(k, carry):
        j0 = 2 * k
        wait(buf0, sem0)
        start(j0 + 1, buf1, sem1)
        pltpu.sync_copy(buf0, out_hbm.at[pl.ds(base + j0 * _G_CH, _G_CH)])

        j1 = j0 + 1
        wait(buf1, sem1)

        @pl.when(j1 + 1 < _G_NCH)
        def _():
            start(j1 + 1, buf0, sem0)

        pltpu.sync_copy(buf1, out_hbm.at[pl.ds(base + j1 * _G_CH, _G_CH)])
        return carry

    lax.fori_loop(0, _G_NCH // 2, chunk, 0)


def _sc_gather(x2, slot_token):
    return pl.kernel(
        _gather_body,
        out_type=jax.ShapeDtypeStruct((NP, D2), jnp.int32),
        mesh=plsc.VectorSubcoreMesh(**_SC_MESH),
        scratch_types=[
            pltpu.VMEM((_G_ROWS,), jnp.int32),
            pltpu.VMEM((_G_CH, D2), jnp.int32),
            pltpu.VMEM((_G_CH, D2), jnp.int32),
            pltpu.SemaphoreType.DMA,
            pltpu.SemaphoreType.DMA,
        ],
    )(x2, slot_token)


def _h_body(te_ref, x_ref, g_ref, u_ref, h_ref):
    xb = x_ref[...]
    gw = g_ref[0].astype(jnp.bfloat16)
    uw = u_ref[0].astype(jnp.bfloat16)
    g = jnp.dot(xb, gw, preferred_element_type=jnp.float32)
    u = jnp.dot(xb, uw, preferred_element_type=jnp.float32)
    h = jax.nn.gelu(g, approximate=True) * u
    h_ref[...] = h.astype(jnp.bfloat16)


def _tc_h(x_g, gate_up_proj, tile_expert):
    grid_spec = pltpu.PrefetchScalarGridSpec(
        num_scalar_prefetch=1,
        grid=(NN, NT),
        in_specs=[
            pl.BlockSpec((TM, D), lambda n, i, te: (i, 0)),
            pl.BlockSpec((1, D, NB), lambda n, i, te: (te[i], 0, n)),
            pl.BlockSpec((1, D, NB), lambda n, i, te: (te[i], 0, NN + n)),
        ],
        out_specs=pl.BlockSpec((TM, NB), lambda n, i, te: (i, n)),
    )
    return pl.pallas_call(
        _h_body,
        grid_spec=grid_spec,
        out_shape=jax.ShapeDtypeStruct((NP, DI), jnp.bfloat16),
        compiler_params=pltpu.CompilerParams(
            dimension_semantics=("arbitrary", "arbitrary")),
    )(tile_expert, x_g, gate_up_proj, gate_up_proj)


DM = D // 2


def _down_body(te_ref, h_ref, d_ref, w_ref, out_ref):
    hb = h_ref[...]
    dw = d_ref[0].astype(jnp.bfloat16)
    p = jnp.dot(hb, dw, preferred_element_type=jnp.float32)
    out_ref[...] = p * w_ref[0, 0, :][:, None]


def _tc_down(h_g, down_proj, slot_weight, tile_expert):
    w3 = slot_weight.reshape(NT, 1, TM)
    grid_spec = pltpu.PrefetchScalarGridSpec(
        num_scalar_prefetch=1,
        grid=(2, NT),
        in_specs=[
            pl.BlockSpec((TM, DI), lambda m, i, te: (i, 0)),
            pl.BlockSpec((1, DI, DM), lambda m, i, te: (te[i], 0, m)),
            pl.BlockSpec((1, 1, TM), lambda m, i, te: (i, 0, 0)),
        ],
        out_specs=pl.BlockSpec((TM, DM), lambda m, i, te: (i, m)),
    )
    return pl.pallas_call(
        _down_body,
        grid_spec=grid_spec,
        out_shape=jax.ShapeDtypeStruct((NP, D), jnp.float32),
        compiler_params=pltpu.CompilerParams(
            dimension_semantics=("arbitrary", "arbitrary")),
    )(tile_expert, h_g, down_proj, w3)


def _tc_ffn(x_g, gate_up_proj, down_proj, slot_weight, tile_expert):
    h_g = _tc_h(x_g, gate_up_proj, tile_expert)
    return _tc_down(h_g, down_proj, slot_weight, tile_expert)


_C_CH = 16
_C_TOK = T // NW
_VR = D // 16


def _combine_body(hg_hbm, p0_hbm, p1_hbm, out_hbm,
                  i0_v, i1_v, r0_v, r1_v, s0, s1):
    wid = lax.axis_index("s") * NC + lax.axis_index("c")
    base = wid * _C_TOK

    def chunk(i, carry):
        off = base + i * _C_CH
        pltpu.sync_copy(p0_hbm.at[pl.ds(off, _C_CH)], i0_v)
        pltpu.sync_copy(p1_hbm.at[pl.ds(off, _C_CH)], i1_v)
        c0 = pltpu.async_copy(hg_hbm.at[i0_v], r0_v, s0)
        c1 = pltpu.async_copy(hg_hbm.at[i1_v], r1_v, s1)
        c0.wait()
        c1.wait()

        def row(r, carry2):
            def vec(j, carry3):
                sl = pl.ds(j * 16, 16)
                r0_v[r, sl] = r0_v[r, sl] + r1_v[r, sl]
                return carry3
            return lax.fori_loop(0, _VR, vec, carry2, unroll=8)

        lax.fori_loop(0, _C_CH, row, 0)
        pltpu.sync_copy(r0_v, out_hbm.at[pl.ds(off, _C_CH)])
        return carry

    lax.fori_loop(0, _C_TOK // _C_CH, chunk, 0)


def _sc_combine(h_g, inv_pos):
    p0 = inv_pos[:, 0]
    p1 = inv_pos[:, 1]
    return pl.kernel(
        _combine_body,
        out_type=jax.ShapeDtypeStruct((T, D), jnp.float32),
        mesh=plsc.VectorSubcoreMesh(**_SC_MESH),
        scratch_types=[
            pltpu.VMEM((_C_CH,), jnp.int32),
            pltpu.VMEM((_C_CH,), jnp.int32),
            pltpu.VMEM((_C_CH, D), jnp.float32),
            pltpu.VMEM((_C_CH, D), jnp.float32),
            pltpu.SemaphoreType.DMA,
            pltpu.SemaphoreType.DMA,
        ],
    )(h_g, p0, p1)


def kernel(x, top_k_index, top_k_weights, gate_up_proj, down_proj):
    slot_token, slot_weight, inv_pos, tile_expert = _routing(
        top_k_index, top_k_weights)
    x2 = lax.bitcast_convert_type(
        x.astype(jnp.bfloat16).reshape(T, D2, 2), jnp.int32)
    xg2 = _sc_gather(x2, slot_token)
    x_g = lax.bitcast_convert_type(xg2, jnp.bfloat16).reshape(NP, D)
    h_g = _tc_ffn(x_g, gate_up_proj, down_proj, slot_weight, tile_expert)
    return _sc_combine(h_g, inv_pos)


# speedup vs baseline: 1.6489x; 1.4475x over previous
import functools

import jax
import jax.numpy as jnp
from jax import lax
from jax.experimental import pallas as pl
from jax.experimental.pallas import tpu as pltpu
from jax.experimental.pallas import tpu_sc as plsc

E = 8
D = 2048
DI = 4096
T = 4096
K = 2
S = T * K

TM = 256
NP = S + E * TM
NT = NP // TM
NB = 1024
NN = DI // NB

NC, NS = 2, 16
NW = NC * NS

_SC_MESH = dict(core_axis_name="c", subcore_axis_name="s",
                num_cores=NC, num_subcores=NS)


def _routing(top_k_index, top_k_weights):
    expert = top_k_index.reshape(-1).astype(jnp.int32)
    token = jnp.arange(S, dtype=jnp.int32) // K
    occ = (expert[:, None] == jnp.arange(E, dtype=jnp.int32)[None, :]
           ).astype(jnp.int32)
    ranks = jnp.cumsum(occ, axis=0) - occ
    counts = jnp.sum(occ, axis=0)
    padded = ((counts + TM - 1) // TM) * TM
    padded_off = jnp.concatenate(
        [jnp.zeros(1, jnp.int32), jnp.cumsum(padded)]).astype(jnp.int32)
    rank = jnp.take_along_axis(ranks, expert[:, None], axis=1)[:, 0]
    pos = padded_off[expert] + rank
    slot_token = jnp.zeros(NP, jnp.int32).at[pos].set(token)
    slot_weight = jnp.zeros(NP, jnp.float32).at[pos].set(
        top_k_weights.reshape(-1))
    inv_pos = pos.reshape(T, K)
    tile_expert = jnp.searchsorted(
        padded_off, jnp.arange(NT, dtype=jnp.int32) * TM,
        side="right").astype(jnp.int32) - 1
    tile_expert = jnp.clip(tile_expert, 0, E - 1)
    return slot_token, slot_weight, inv_pos, tile_expert


_G_CH = 8
_G_ROWS = NP // NW
_G_NCH = _G_ROWS // _G_CH
_G_NBUF = 4


def _gather_body(x_hbm, idx_hbm, out_hbm, idx_v, *bufs_sems):
    bufs = bufs_sems[:_G_NBUF]
    gsem = bufs_sems[_G_NBUF:2 * _G_NBUF]
    wsem = bufs_sems[2 * _G_NBUF:3 * _G_NBUF]
    wid = lax.axis_index("s") * NC + lax.axis_index("c")
    base = wid * _G_ROWS
    pltpu.sync_copy(idx_hbm.at[pl.ds(base, _G_ROWS)], idx_v)

    def g_start(j, b):
        pltpu.make_async_copy(
            x_hbm.at[idx_v.at[pl.ds(j * _G_CH, _G_CH)]],
            bufs[b], gsem[b]).start()

    def g_wait(b):
        pltpu.make_async_copy(
            x_hbm.at[idx_v.at[pl.ds(0, _G_CH)]], bufs[b], gsem[b]).wait()

    def w_start(j, b):
        pltpu.make_async_copy(
            bufs[b], out_hbm.at[pl.ds(base + j * _G_CH, _G_CH)],
            wsem[b]).start()

    def w_wait(b):
        pltpu.make_async_copy(
            bufs[b], out_hbm.at[pl.ds(base, _G_CH)], wsem[b]).wait()

    for b in range(_G_NBUF):
        g_start(b, b)

    def ring(k, carry):
        for b in range(_G_NBUF):
            j = _G_NBUF * k + b
            g_wait(b)
            w_start(j, b)
        for b in range(_G_NBUF):
            jn = _G_NBUF * (k + 1) + b
            w_wait(b)

            @pl.when(jn < _G_NCH)
            def _():
                g_start(jn, b)
        return carry

    lax.fori_loop(0, _G_NCH // _G_NBUF, ring, 0)


def _sc_gather(x, slot_token):
    return pl.kernel(
        _gather_body,
        out_type=jax.ShapeDtypeStruct((NP, D), jnp.float32),
        mesh=plsc.VectorSubcoreMesh(**_SC_MESH),
        scratch_types=(
            [pltpu.VMEM((_G_ROWS,), jnp.int32)]
            + [pltpu.VMEM((_G_CH, D), jnp.float32) for _ in range(_G_NBUF)]
            + [pltpu.SemaphoreType.DMA] * (2 * _G_NBUF)
        ),
    )(x, slot_token)


def _h_body(te_ref, x_ref, g_ref, u_ref, h_ref):
    xb = x_ref[...].astype(jnp.bfloat16)
    gw = g_ref[0].astype(jnp.bfloat16)
    uw = u_ref[0].astype(jnp.bfloat16)
    g = jnp.dot(xb, gw, preferred_element_type=jnp.float32)
    u = jnp.dot(xb, uw, preferred_element_type=jnp.float32)
    h = jax.nn.gelu(g, approximate=True) * u
    h_ref[...] = h.astype(jnp.bfloat16)


def _tc_h(x_g, gate_up_proj, tile_expert):
    grid_spec = pltpu.PrefetchScalarGridSpec(
        num_scalar_prefetch=1,
        grid=(NN, NT),
        in_specs=[
            pl.BlockSpec((TM, D), lambda n, i, te: (i, 0)),
            pl.BlockSpec((1, D, NB), lambda n, i, te: (te[i], 0, n)),
            pl.BlockSpec((1, D, NB), lambda n, i, te: (te[i], 0, NN + n)),
        ],
        out_specs=pl.BlockSpec((TM, NB), lambda n, i, te: (i, n)),
    )
    return pl.pallas_call(
        _h_body,
        grid_spec=grid_spec,
        out_shape=jax.ShapeDtypeStruct((NP, DI), jnp.bfloat16),
        compiler_params=pltpu.CompilerParams(
            dimension_semantics=("arbitrary", "arbitrary")),
    )(tile_expert, x_g, gate_up_proj, gate_up_proj)


DM = D // 2


def _down_body(te_ref, h_ref, d_ref, w_ref, out_ref):
    hb = h_ref[...]
    dw = d_ref[0].astype(jnp.bfloat16)
    p = jnp.dot(hb, dw, preferred_element_type=jnp.float32)
    out_ref[...] = p * w_ref[0, 0, :][:, None]


def _tc_down(h_g, down_proj, slot_weight, tile_expert):
    w3 = slot_weight.reshape(NT, 1, TM)
    grid_spec = pltpu.PrefetchScalarGridSpec(
        num_scalar_prefetch=1,
        grid=(2, NT),
        in_specs=[
            pl.BlockSpec((TM, DI), lambda m, i, te: (i, 0)),
            pl.BlockSpec((1, DI, DM), lambda m, i, te: (te[i], 0, m)),
            pl.BlockSpec((1, 1, TM), lambda m, i, te: (i, 0, 0)),
        ],
        out_specs=pl.BlockSpec((TM, DM), lambda m, i, te: (i, m)),
    )
    return pl.pallas_call(
        _down_body,
        grid_spec=grid_spec,
        out_shape=jax.ShapeDtypeStruct((NP, D), jnp.float32),
        compiler_params=pltpu.CompilerParams(
            dimension_semantics=("arbitrary", "arbitrary")),
    )(tile_expert, h_g, down_proj, w3)


def _tc_ffn(x_g, gate_up_proj, down_proj, slot_weight, tile_expert):
    h_g = _tc_h(x_g, gate_up_proj, tile_expert)
    return _tc_down(h_g, down_proj, slot_weight, tile_expert)


_C_CH = 8
_C_TOK = T // NW
_C_NCH = _C_TOK // _C_CH
_VR = D // 16


def _combine_body(hg_hbm, p0_hbm, p1_hbm, out_hbm, i0_v, i1_v,
                  a0, b0, a1, b1, ga0, gb0, ga1, gb1, w0, w1):
    wid = lax.axis_index("s") * NC + lax.axis_index("c")
    base = wid * _C_TOK
    pltpu.sync_copy(p0_hbm.at[pl.ds(base, _C_TOK)], i0_v)
    pltpu.sync_copy(p1_hbm.at[pl.ds(base, _C_TOK)], i1_v)

    bufs = ((a0, b0, ga0, gb0, w0), (a1, b1, ga1, gb1, w1))

    def g_start(j, p):
        a, b, ga, gb, _ = bufs[p]
        sl = pl.ds(j * _C_CH, _C_CH)
        pltpu.make_async_copy(hg_hbm.at[i0_v.at[sl]], a, ga).start()
        pltpu.make_async_copy(hg_hbm.at[i1_v.at[sl]], b, gb).start()

    def g_wait(p):
        a, b, ga, gb, _ = bufs[p]
        sl = pl.ds(0, _C_CH)
        pltpu.make_async_copy(hg_hbm.at[i0_v.at[sl]], a, ga).wait()
        pltpu.make_async_copy(hg_hbm.at[i1_v.at[sl]], b, gb).wait()

    def add_rows(p):
        a, b, _, _, _ = bufs[p]

        def row(r, carry2):
            def vec(j, carry3):
                sl = pl.ds(j * 16, 16)
                a[r, sl] = a[r, sl] + b[r, sl]
                return carry3
            return lax.fori_loop(0, _VR, vec, carry2, unroll=8)

        lax.fori_loop(0, _C_CH, row, 0)

    def w_start(j, p):
        a, _, _, _, w = bufs[p]
        pltpu.make_async_copy(
            a, out_hbm.at[pl.ds(base + j * _C_CH, _C_CH)], w).start()

    def w_wait(p):
        a, _, _, _, w = bufs[p]
        pltpu.make_async_copy(a, out_hbm.at[pl.ds(base, _C_CH)], w).wait()

    g_start(0, 0)
    g_start(1, 1)

    def chunk(k, carry):
        for p in range(2):
            j = 2 * k + p
            g_wait(p)
            add_rows(p)
            w_start(j, p)
        for p in range(2):
            jn = 2 * (k + 1) + p
            w_wait(p)

            @pl.when(jn < _C_NCH)
            def _():
                g_start(jn, p)
        return carry

    lax.fori_loop(0, _C_NCH // 2, chunk, 0)


def _sc_combine(h_g, inv_pos):
    p0 = inv_pos[:, 0]
    p1 = inv_pos[:, 1]
    return pl.kernel(
        _combine_body,
        out_type=jax.ShapeDtypeStruct((T, D), jnp.float32),
        mesh=plsc.VectorSubcoreMesh(**_SC_MESH),
        scratch_types=(
            [pltpu.VMEM((_C_TOK,), jnp.int32)] * 2
            + [pltpu.VMEM((_C_CH, D), jnp.float32) for _ in range(4)]
            + [pltpu.SemaphoreType.DMA] * 6
        ),
    )(h_g, p0, p1)


def kernel(x, top_k_index, top_k_weights, gate_up_proj, down_proj):
    slot_token, slot_weight, inv_pos, tile_expert = _routing(
        top_k_index, top_k_weights)
    x_g = _sc_gather(x, slot_token)
    h_g = _tc_ffn(x_g, gate_up_proj, down_proj, slot_weight, tile_expert)
    return _sc_combine(h_g, inv_pos)


# speedup vs baseline: 1.6665x; 1.0107x over previous
import functools

import jax
import jax.numpy as jnp
from jax import lax
from jax.experimental import pallas as pl
from jax.experimental.pallas import tpu as pltpu
from jax.experimental.pallas import tpu_sc as plsc

E = 8
D = 2048
DI = 4096
T = 4096
K = 2
S = T * K

TM = 256
NP = S + E * TM
NT = NP // TM
NB = 1024
NN = DI // NB

NC, NS = 2, 16
NW = NC * NS

_SC_MESH = dict(core_axis_name="c", subcore_axis_name="s",
                num_cores=NC, num_subcores=NS)


def _routing(top_k_index, top_k_weights):
    expert = top_k_index.reshape(-1).astype(jnp.int32)
    token = jnp.arange(S, dtype=jnp.int32) // K
    occ = (expert[:, None] == jnp.arange(E, dtype=jnp.int32)[None, :]
           ).astype(jnp.int32)
    ranks = jnp.cumsum(occ, axis=0) - occ
    counts = jnp.sum(occ, axis=0)
    padded = ((counts + TM - 1) // TM) * TM
    padded_off = jnp.concatenate(
        [jnp.zeros(1, jnp.int32), jnp.cumsum(padded)]).astype(jnp.int32)
    rank = jnp.take_along_axis(ranks, expert[:, None], axis=1)[:, 0]
    pos = padded_off[expert] + rank
    slot_token = jnp.zeros(NP, jnp.int32).at[pos].set(token)
    slot_weight = jnp.zeros(NP, jnp.float32).at[pos].set(
        top_k_weights.reshape(-1))
    inv_pos = pos.reshape(T, K)
    tile_expert = jnp.searchsorted(
        padded_off, jnp.arange(NT, dtype=jnp.int32) * TM,
        side="right").astype(jnp.int32) - 1
    tile_expert = jnp.clip(tile_expert, 0, E - 1)
    return slot_token, slot_weight, inv_pos, tile_expert


_G_CH = 16
_G_ROWS = NP // NW
_G_NCH = _G_ROWS // _G_CH
_G_NBUF = 2


def _gather_body(x_hbm, idx_hbm, out_hbm, idx_v, *bufs_sems):
    bufs = bufs_sems[:_G_NBUF]
    gsem = bufs_sems[_G_NBUF:2 * _G_NBUF]
    wsem = bufs_sems[2 * _G_NBUF:3 * _G_NBUF]
    wid = lax.axis_index("s") * NC + lax.axis_index("c")
    base = wid * _G_ROWS
    pltpu.sync_copy(idx_hbm.at[pl.ds(base, _G_ROWS)], idx_v)

    def g_start(j, b):
        pltpu.make_async_copy(
            x_hbm.at[idx_v.at[pl.ds(j * _G_CH, _G_CH)]],
            bufs[b], gsem[b]).start()

    def g_wait(b):
        pltpu.make_async_copy(
            x_hbm.at[idx_v.at[pl.ds(0, _G_CH)]], bufs[b], gsem[b]).wait()

    def w_start(j, b):
        pltpu.make_async_copy(
            bufs[b], out_hbm.at[pl.ds(base + j * _G_CH, _G_CH)],
            wsem[b]).start()

    def w_wait(b):
        pltpu.make_async_copy(
            bufs[b], out_hbm.at[pl.ds(base, _G_CH)], wsem[b]).wait()

    for b in range(_G_NBUF):
        g_start(b, b)

    def ring(k, carry):
        for b in range(_G_NBUF):
            j = _G_NBUF * k + b
            g_wait(b)
            w_start(j, b)
        for b in range(_G_NBUF):
            jn = _G_NBUF * (k + 1) + b
            w_wait(b)

            @pl.when(jn < _G_NCH)
            def _():
                g_start(jn, b)
        return carry

    lax.fori_loop(0, _G_NCH // _G_NBUF, ring, 0)


def _sc_gather(x, slot_token):
    return pl.kernel(
        _gather_body,
        out_type=jax.ShapeDtypeStruct((NP, D), jnp.float32),
        mesh=plsc.VectorSubcoreMesh(**_SC_MESH),
        scratch_types=(
            [pltpu.VMEM((_G_ROWS,), jnp.int32)]
            + [pltpu.VMEM((_G_CH, D), jnp.float32) for _ in range(_G_NBUF)]
            + [pltpu.SemaphoreType.DMA] * (2 * _G_NBUF)
        ),
    )(x, slot_token)


def _h_body(te_ref, x_ref, g_ref, u_ref, h_ref):
    xb = x_ref[...]
    g = jnp.dot(xb, g_ref[0], preferred_element_type=jnp.float32)
    u = jnp.dot(xb, u_ref[0], preferred_element_type=jnp.float32)
    h = jax.nn.gelu(g, approximate=True) * u
    h_ref[...] = h.astype(jnp.bfloat16)


def _tc_h(x_g, gate_up_proj, tile_expert):
    grid_spec = pltpu.PrefetchScalarGridSpec(
        num_scalar_prefetch=1,
        grid=(NN, NT),
        in_specs=[
            pl.BlockSpec((TM, D), lambda n, i, te: (i, 0)),
            pl.BlockSpec((1, D, NB), lambda n, i, te: (te[i], 0, n)),
            pl.BlockSpec((1, D, NB), lambda n, i, te: (te[i], 0, NN + n)),
        ],
        out_specs=pl.BlockSpec((TM, NB), lambda n, i, te: (i, n)),
    )
    return pl.pallas_call(
        _h_body,
        grid_spec=grid_spec,
        out_shape=jax.ShapeDtypeStruct((NP, DI), jnp.bfloat16),
        compiler_params=pltpu.CompilerParams(
            dimension_semantics=("arbitrary", "arbitrary")),
    )(tile_expert, x_g, gate_up_proj, gate_up_proj)


DM = D // 2


def _down_body(te_ref, h_ref, d_ref, w_ref, out_ref):
    hb = h_ref[...]
    dw = d_ref[0].astype(jnp.bfloat16)
    p = jnp.dot(hb, dw, preferred_element_type=jnp.float32)
    out_ref[...] = p * w_ref[0, 0, :][:, None]


def _tc_down(h_g, down_proj, slot_weight, tile_expert):
    w3 = slot_weight.reshape(NT, 1, TM)
    grid_spec = pltpu.PrefetchScalarGridSpec(
        num_scalar_prefetch=1,
        grid=(2, NT),
        in_specs=[
            pl.BlockSpec((TM, DI), lambda m, i, te: (i, 0)),
            pl.BlockSpec((1, DI, DM), lambda m, i, te: (te[i], 0, m)),
            pl.BlockSpec((1, 1, TM), lambda m, i, te: (i, 0, 0)),
        ],
        out_specs=pl.BlockSpec((TM, DM), lambda m, i, te: (i, m)),
    )
    return pl.pallas_call(
        _down_body,
        grid_spec=grid_spec,
        out_shape=jax.ShapeDtypeStruct((NP, D), jnp.float32),
        compiler_params=pltpu.CompilerParams(
            dimension_semantics=("arbitrary", "arbitrary")),
    )(tile_expert, h_g, down_proj, w3)


def _tc_ffn(x_g, gate_up_proj, down_proj, slot_weight, tile_expert):
    h_g = _tc_h(x_g, gate_up_proj, tile_expert)
    return _tc_down(h_g, down_proj, slot_weight, tile_expert)


_C_CH = 8
_C_TOK = T // NW
_C_NCH = _C_TOK // _C_CH
_VR = D // 16


def _combine_body(hg_hbm, p0_hbm, p1_hbm, out_hbm, i0_v, i1_v,
                  a0, b0, a1, b1, ga0, gb0, ga1, gb1, w0, w1):
    wid = lax.axis_index("s") * NC + lax.axis_index("c")
    base = wid * _C_TOK
    pltpu.sync_copy(p0_hbm.at[pl.ds(base, _C_TOK)], i0_v)
    pltpu.sync_copy(p1_hbm.at[pl.ds(base, _C_TOK)], i1_v)

    bufs = ((a0, b0, ga0, gb0, w0), (a1, b1, ga1, gb1, w1))

    def g_start(j, p):
        a, b, ga, gb, _ = bufs[p]
        sl = pl.ds(j * _C_CH, _C_CH)
        pltpu.make_async_copy(hg_hbm.at[i0_v.at[sl]], a, ga).start()
        pltpu.make_async_copy(hg_hbm.at[i1_v.at[sl]], b, gb).start()

    def g_wait(p):
        a, b, ga, gb, _ = bufs[p]
        sl = pl.ds(0, _C_CH)
        pltpu.make_async_copy(hg_hbm.at[i0_v.at[sl]], a, ga).wait()
        pltpu.make_async_copy(hg_hbm.at[i1_v.at[sl]], b, gb).wait()

    def add_rows(p):
        a, b, _, _, _ = bufs[p]

        def row(r, carry2):
            def vec(j, carry3):
                sl = pl.ds(j * 16, 16)
                a[r, sl] = a[r, sl] + b[r, sl]
                return carry3
            return lax.fori_loop(0, _VR, vec, carry2, unroll=8)

        lax.fori_loop(0, _C_CH, row, 0)

    def w_start(j, p):
        a, _, _, _, w = bufs[p]
        pltpu.make_async_copy(
            a, out_hbm.at[pl.ds(base + j * _C_CH, _C_CH)], w).start()

    def w_wait(p):
        a, _, _, _, w = bufs[p]
        pltpu.make_async_copy(a, out_hbm.at[pl.ds(base, _C_CH)], w).wait()

    g_start(0, 0)
    g_start(1, 1)

    def chunk(k, carry):
        for p in range(2):
            j = 2 * k + p
            g_wait(p)
            add_rows(p)
            w_start(j, p)
        for p in range(2):
            jn = 2 * (k + 1) + p
            w_wait(p)

            @pl.when(jn < _C_NCH)
            def _():
                g_start(jn, p)
        return carry

    lax.fori_loop(0, _C_NCH // 2, chunk, 0)


def _sc_combine(h_g, inv_pos):
    p0 = inv_pos[:, 0]
    p1 = inv_pos[:, 1]
    return pl.kernel(
        _combine_body,
        out_type=jax.ShapeDtypeStruct((T, D), jnp.float32),
        mesh=plsc.VectorSubcoreMesh(**_SC_MESH),
        scratch_types=(
            [pltpu.VMEM((_C_TOK,), jnp.int32)] * 2
            + [pltpu.VMEM((_C_CH, D), jnp.float32) for _ in range(4)]
            + [pltpu.SemaphoreType.DMA] * 6
        ),
    )(h_g, p0, p1)


def kernel(x, top_k_index, top_k_weights, gate_up_proj, down_proj):
    slot_token, slot_weight, inv_pos, tile_expert = _routing(
        top_k_index, top_k_weights)
    x_g = _sc_gather(x, slot_token)
    h_g = _tc_ffn(x_g, gate_up_proj, down_proj, slot_weight, tile_expert)
    return _sc_combine(h_g, inv_pos)


# speedup vs baseline: 1.7325x; 1.0396x over previous
import functools

import jax
import jax.numpy as jnp
from jax import lax
from jax.experimental import pallas as pl
from jax.experimental.pallas import tpu as pltpu
from jax.experimental.pallas import tpu_sc as plsc

E = 8
D = 2048
DI = 4096
T = 4096
K = 2
S = T * K

TM = 256
NP = S + E * TM
NT = NP // TM
NB = 1024
NN = DI // NB

NC, NS = 2, 16
NW = NC * NS

_SC_MESH = dict(core_axis_name="c", subcore_axis_name="s",
                num_cores=NC, num_subcores=NS)


def _routing(top_k_index, top_k_weights):
    expert = top_k_index.reshape(-1).astype(jnp.int32)
    token = jnp.arange(S, dtype=jnp.int32) // K
    occ = (expert[:, None] == jnp.arange(E, dtype=jnp.int32)[None, :]
           ).astype(jnp.int32)
    ranks = jnp.cumsum(occ, axis=0) - occ
    counts = jnp.sum(occ, axis=0)
    padded = ((counts + TM - 1) // TM) * TM
    padded_off = jnp.concatenate(
        [jnp.zeros(1, jnp.int32), jnp.cumsum(padded)]).astype(jnp.int32)
    rank = jnp.take_along_axis(ranks, expert[:, None], axis=1)[:, 0]
    pos = padded_off[expert] + rank
    slot_token = jnp.zeros(NP, jnp.int32).at[pos].set(token)
    slot_weight = jnp.zeros(NP, jnp.float32).at[pos].set(
        top_k_weights.reshape(-1))
    inv_pos = pos.reshape(T, K)
    tile_expert = jnp.searchsorted(
        padded_off, jnp.arange(NT, dtype=jnp.int32) * TM,
        side="right").astype(jnp.int32) - 1
    tile_expert = jnp.clip(tile_expert, 0, E - 1)
    tile_live = (jnp.arange(NT, dtype=jnp.int32) * TM
                 < padded_off[E]).astype(jnp.int32)
    return slot_token, slot_weight, inv_pos, tile_expert, tile_live


_G_CH = 16
_G_ROWS = (NP // 2) // NW
_G_NCH = _G_ROWS // _G_CH
_G_NBUF = 2


def _gather_body(x_hbm, idx_hbm, out_hbm, idx_v, *bufs_sems):
    bufs = bufs_sems[:_G_NBUF]
    gsem = bufs_sems[_G_NBUF:2 * _G_NBUF]
    wsem = bufs_sems[2 * _G_NBUF:3 * _G_NBUF]
    wid = lax.axis_index("s") * NC + lax.axis_index("c")
    base = wid * _G_ROWS
    pltpu.sync_copy(idx_hbm.at[pl.ds(base, _G_ROWS)], idx_v)

    def g_start(j, b):
        pltpu.make_async_copy(
            x_hbm.at[idx_v.at[pl.ds(j * _G_CH, _G_CH)]],
            bufs[b], gsem[b]).start()

    def g_wait(b):
        pltpu.make_async_copy(
            x_hbm.at[idx_v.at[pl.ds(0, _G_CH)]], bufs[b], gsem[b]).wait()

    def w_start(j, b):
        pltpu.make_async_copy(
            bufs[b], out_hbm.at[pl.ds(base + j * _G_CH, _G_CH)],
            wsem[b]).start()

    def w_wait(b):
        pltpu.make_async_copy(
            bufs[b], out_hbm.at[pl.ds(base, _G_CH)], wsem[b]).wait()

    for b in range(_G_NBUF):
        g_start(b, b)

    def ring(k, carry):
        for b in range(_G_NBUF):
            j = _G_NBUF * k + b
            g_wait(b)
            w_start(j, b)
        for b in range(_G_NBUF):
            jn = _G_NBUF * (k + 1) + b
            w_wait(b)

            @pl.when(jn < _G_NCH)
            def _():
                g_start(jn, b)
        return carry

    lax.fori_loop(0, _G_NCH // _G_NBUF, ring, 0)


def _sc_gather(x, slot_token_half):
    return pl.kernel(
        _gather_body,
        out_type=jax.ShapeDtypeStruct((NP // 2, D), jnp.float32),
        mesh=plsc.VectorSubcoreMesh(**_SC_MESH),
        scratch_types=(
            [pltpu.VMEM((_G_ROWS,), jnp.int32)]
            + [pltpu.VMEM((_G_CH, D), jnp.float32) for _ in range(_G_NBUF)]
            + [pltpu.SemaphoreType.DMA] * (2 * _G_NBUF)
        ),
    )(x, slot_token_half)


NT2 = NT // 2


def _h_body(te_ref, lv_ref, x_ref, g_ref, u_ref, hp_ref, h_ref):
    del hp_ref
    i = pl.program_id(1)

    @pl.when(lv_ref[i] > 0)
    def _():
        xb = x_ref[...]
        g = jnp.dot(xb, g_ref[0], preferred_element_type=jnp.float32)
        u = jnp.dot(xb, u_ref[0], preferred_element_type=jnp.float32)
        h = jax.nn.gelu(g, approximate=True) * u
        h_ref[...] = h.astype(jnp.bfloat16)


def _h_body0(te_ref, lv_ref, x_ref, g_ref, u_ref, h_ref):
    _h_body(te_ref, lv_ref, x_ref, g_ref, u_ref, None, h_ref)


def _tc_h(x_half, gate_up_proj, te_half, lv_half, half, h_prev=None):
    base = half * NT2
    in_specs = [
        pl.BlockSpec((TM, D), lambda n, i, te, lv: (i, 0)),
        pl.BlockSpec((1, D, NB), lambda n, i, te, lv: (te[i], 0, n)),
        pl.BlockSpec((1, D, NB), lambda n, i, te, lv: (te[i], 0, NN + n)),
    ]
    args = [te_half, lv_half, x_half, gate_up_proj, gate_up_proj]
    if h_prev is None:
        body, aliases = _h_body0, {}
    else:
        body, aliases = _h_body, {5: 0}
        in_specs = in_specs + [
            pl.BlockSpec(memory_space=pltpu.MemorySpace.HBM)]
        args = args + [h_prev]
    grid_spec = pltpu.PrefetchScalarGridSpec(
        num_scalar_prefetch=2,
        grid=(NN, NT2),
        in_specs=in_specs,
        out_specs=pl.BlockSpec((TM, NB), lambda n, i, te, lv: (base + i, n)),
    )
    return pl.pallas_call(
        body,
        grid_spec=grid_spec,
        out_shape=jax.ShapeDtypeStruct((NP, DI), jnp.bfloat16),
        input_output_aliases=aliases,
        compiler_params=pltpu.CompilerParams(
            dimension_semantics=("arbitrary", "arbitrary")),
    )(*args)


DM = D // 2


def _down_body(te_ref, lv_ref, h_ref, d_ref, w_ref, out_ref):
    i = pl.program_id(1)

    @pl.when(lv_ref[i] > 0)
    def _():
        hb = h_ref[...]
        dw = d_ref[0].astype(jnp.bfloat16)
        p = jnp.dot(hb, dw, preferred_element_type=jnp.float32)
        out_ref[...] = p * w_ref[0, 0, :][:, None]


def _tc_down(h_g, down_proj, slot_weight, tile_expert, tile_live):
    w3 = slot_weight.reshape(NT, 1, TM)
    grid_spec = pltpu.PrefetchScalarGridSpec(
        num_scalar_prefetch=2,
        grid=(2, NT),
        in_specs=[
            pl.BlockSpec((TM, DI), lambda m, i, te, lv: (i, 0)),
            pl.BlockSpec((1, DI, DM), lambda m, i, te, lv: (te[i], 0, m)),
            pl.BlockSpec((1, 1, TM), lambda m, i, te, lv: (i, 0, 0)),
        ],
        out_specs=pl.BlockSpec((TM, DM), lambda m, i, te, lv: (i, m)),
    )
    return pl.pallas_call(
        _down_body,
        grid_spec=grid_spec,
        out_shape=jax.ShapeDtypeStruct((NP, D), jnp.float32),
        compiler_params=pltpu.CompilerParams(
            dimension_semantics=("arbitrary", "arbitrary")),
    )(tile_expert, tile_live, h_g, down_proj, w3)


_C_CH = 8
_C_TOK = T // NW
_C_NCH = _C_TOK // _C_CH
_VR = D // 16


def _combine_body(hg_hbm, p0_hbm, p1_hbm, out_hbm, i0_v, i1_v,
                  a0, b0, a1, b1, ga0, gb0, ga1, gb1, w0, w1):
    wid = lax.axis_index("s") * NC + lax.axis_index("c")
    base = wid * _C_TOK
    pltpu.sync_copy(p0_hbm.at[pl.ds(base, _C_TOK)], i0_v)
    pltpu.sync_copy(p1_hbm.at[pl.ds(base, _C_TOK)], i1_v)

    bufs = ((a0, b0, ga0, gb0, w0), (a1, b1, ga1, gb1, w1))

    def g_start(j, p):
        a, b, ga, gb, _ = bufs[p]
        sl = pl.ds(j * _C_CH, _C_CH)
        pltpu.make_async_copy(hg_hbm.at[i0_v.at[sl]], a, ga).start()
        pltpu.make_async_copy(hg_hbm.at[i1_v.at[sl]], b, gb).start()

    def g_wait(p):
        a, b, ga, gb, _ = bufs[p]
        sl = pl.ds(0, _C_CH)
        pltpu.make_async_copy(hg_hbm.at[i0_v.at[sl]], a, ga).wait()
        pltpu.make_async_copy(hg_hbm.at[i1_v.at[sl]], b, gb).wait()

    def add_rows(p):
        a, b, _, _, _ = bufs[p]

        def row(r, carry2):
            def vec(j, carry3):
                sl = pl.ds(j * 16, 16)
                a[r, sl] = a[r, sl] + b[r, sl]
                return carry3
            return lax.fori_loop(0, _VR, vec, carry2, unroll=8)

        lax.fori_loop(0, _C_CH, row, 0)

    def w_start(j, p):
        a, _, _, _, w = bufs[p]
        pltpu.make_async_copy(
            a, out_hbm.at[pl.ds(base + j * _C_CH, _C_CH)], w).start()

    def w_wait(p):
        a, _, _, _, w = bufs[p]
        pltpu.make_async_copy(a, out_hbm.at[pl.ds(base, _C_CH)], w).wait()

    g_start(0, 0)
    g_start(1, 1)

    def chunk(k, carry):
        for p in range(2):
            j = 2 * k + p
            g_wait(p)
            add_rows(p)
            w_start(j, p)
        for p in range(2):
            jn = 2 * (k + 1) + p
            w_wait(p)

            @pl.when(jn < _C_NCH)
            def _():
                g_start(jn, p)
        return carry

    lax.fori_loop(0, _C_NCH // 2, chunk, 0)


def _sc_combine(h_g, inv_pos):
    p0 = inv_pos[:, 0]
    p1 = inv_pos[:, 1]
    return pl.kernel(
        _combine_body,
        out_type=jax.ShapeDtypeStruct((T, D), jnp.float32),
        mesh=plsc.VectorSubcoreMesh(**_SC_MESH),
        scratch_types=(
            [pltpu.VMEM((_C_TOK,), jnp.int32)] * 2
            + [pltpu.VMEM((_C_CH, D), jnp.float32) for _ in range(4)]
            + [pltpu.SemaphoreType.DMA] * 6
        ),
    )(h_g, p0, p1)


def kernel(x, top_k_index, top_k_weights, gate_up_proj, down_proj):
    slot_token, slot_weight, inv_pos, tile_expert, tile_live = _routing(
        top_k_index, top_k_weights)
    half = NP // 2
    x_g0 = _sc_gather(x, slot_token[:half])
    x_g1 = _sc_gather(x, slot_token[half:])
    h_g = _tc_h(x_g0, gate_up_proj, tile_expert[:NT2], tile_live[:NT2], 0)
    h_g = _tc_h(x_g1, gate_up_proj, tile_expert[NT2:], tile_live[NT2:],
                1, h_g)
    out_g = _tc_down(h_g, down_proj, slot_weight, tile_expert, tile_live)
    return _sc_combine(out_g, inv_pos)


# speedup vs baseline: 1.7438x; 1.0065x over previous
import functools

import jax
import jax.numpy as jnp
from jax import lax
from jax.experimental import pallas as pl
from jax.experimental.pallas import tpu as pltpu
from jax.experimental.pallas import tpu_sc as plsc

E = 8
D = 2048
DI = 4096
T = 4096
K = 2
S = T * K

TM = 256
NP = S + E * TM
NT = NP // TM
NB = 1024
NN = DI // NB

NC, NS = 2, 16
NW = NC * NS

_SC_MESH = dict(core_axis_name="c", subcore_axis_name="s",
                num_cores=NC, num_subcores=NS)


def _routing(top_k_index, top_k_weights):
    expert = top_k_index.reshape(-1).astype(jnp.int32)
    token = jnp.arange(S, dtype=jnp.int32) // K
    occ = (expert[:, None] == jnp.arange(E, dtype=jnp.int32)[None, :]
           ).astype(jnp.int32)
    ranks = jnp.cumsum(occ, axis=0) - occ
    counts = jnp.sum(occ, axis=0)
    padded = ((counts + TM - 1) // TM) * TM
    padded_off = jnp.concatenate(
        [jnp.zeros(1, jnp.int32), jnp.cumsum(padded)]).astype(jnp.int32)
    rank = jnp.take_along_axis(ranks, expert[:, None], axis=1)[:, 0]
    pos = padded_off[expert] + rank
    slot_token = jnp.zeros(NP, jnp.int32).at[pos].set(token)
    slot_weight = jnp.zeros(NP, jnp.float32).at[pos].set(
        top_k_weights.reshape(-1))
    inv_pos = pos.reshape(T, K)
    tile_expert = jnp.searchsorted(
        padded_off, jnp.arange(NT, dtype=jnp.int32) * TM,
        side="right").astype(jnp.int32) - 1
    tile_expert = jnp.clip(tile_expert, 0, E - 1)
    tile_live = (jnp.arange(NT, dtype=jnp.int32) * TM
                 < padded_off[E]).astype(jnp.int32)
    return slot_token, slot_weight, inv_pos, tile_expert, tile_live


_G_CH = 16
_G_ROWS = (NP // 2) // NW
_G_NCH = _G_ROWS // _G_CH
_G_NBUF = 2


def _gather_body(x_hbm, idx_hbm, out_hbm, idx_v, *bufs_sems):
    bufs = bufs_sems[:_G_NBUF]
    gsem = bufs_sems[_G_NBUF:2 * _G_NBUF]
    wsem = bufs_sems[2 * _G_NBUF:3 * _G_NBUF]
    wid = lax.axis_index("s") * NC + lax.axis_index("c")
    base = wid * _G_ROWS
    pltpu.sync_copy(idx_hbm.at[pl.ds(base, _G_ROWS)], idx_v)

    def g_start(j, b):
        pltpu.make_async_copy(
            x_hbm.at[idx_v.at[pl.ds(j * _G_CH, _G_CH)]],
            bufs[b], gsem[b]).start()

    def g_wait(b):
        pltpu.make_async_copy(
            x_hbm.at[idx_v.at[pl.ds(0, _G_CH)]], bufs[b], gsem[b]).wait()

    def w_start(j, b):
        pltpu.make_async_copy(
            bufs[b], out_hbm.at[pl.ds(base + j * _G_CH, _G_CH)],
            wsem[b]).start()

    def w_wait(b):
        pltpu.make_async_copy(
            bufs[b], out_hbm.at[pl.ds(base, _G_CH)], wsem[b]).wait()

    for b in range(_G_NBUF):
        g_start(b, b)

    def ring(k, carry):
        for b in range(_G_NBUF):
            j = _G_NBUF * k + b
            g_wait(b)
            w_start(j, b)
        for b in range(_G_NBUF):
            jn = _G_NBUF * (k + 1) + b
            w_wait(b)

            @pl.when(jn < _G_NCH)
            def _():
                g_start(jn, b)
        return carry

    lax.fori_loop(0, _G_NCH // _G_NBUF, ring, 0)


def _sc_gather(x, slot_token_half):
    return pl.kernel(
        _gather_body,
        out_type=jax.ShapeDtypeStruct((NP // 2, D), jnp.float32),
        mesh=plsc.VectorSubcoreMesh(**_SC_MESH),
        scratch_types=(
            [pltpu.VMEM((_G_ROWS,), jnp.int32)]
            + [pltpu.VMEM((_G_CH, D), jnp.float32) for _ in range(_G_NBUF)]
            + [pltpu.SemaphoreType.DMA] * (2 * _G_NBUF)
        ),
    )(x, slot_token_half)


NT2 = NT // 2


_LA = 3
_S1 = NN * NT2


def _h_sched(te_half):
    s = jnp.arange(_S1, dtype=jnp.int32)
    i_s = s % NT2
    n_s = s // NT2
    te_s = te_half[i_s]
    prev_te = jnp.concatenate([jnp.full((1,), -1, jnp.int32), te_s[:-1]])
    prev_n = jnp.concatenate([jnp.full((1,), -1, jnp.int32), n_s[:-1]])
    chg = ((te_s != prev_te) | (n_s != prev_n)).astype(jnp.int32)
    blk = jnp.cumsum(chg) - 1
    nblk = blk[-1] + 1
    big = jnp.int32(10 * _S1)
    fu = jnp.full((_S1,), big, jnp.int32).at[blk].min(s)
    kk = jnp.arange(_S1, dtype=jnp.int32)
    fu_prev = jnp.concatenate([jnp.full((1,), -big, jnp.int32), fu[:-1]])
    issue = jnp.maximum(fu - _LA, fu_prev)
    issue = kk + jax.lax.cummax(issue - kk, axis=0)
    issue = jnp.maximum(issue, kk)
    issue = jnp.where(kk < nblk, issue, big)
    fusafe = jnp.clip(fu, 0, _S1 - 1)
    f_te = jnp.full((_S1,), -1, jnp.int32).at[issue].set(
        te_s[fusafe], mode="drop")
    f_n = jnp.zeros((_S1,), jnp.int32).at[issue].set(
        n_s[fusafe], mode="drop")
    f_slot = jnp.zeros((_S1,), jnp.int32).at[issue].set(
        kk % 2, mode="drop")
    cslot = blk % 2
    return f_te, f_n, f_slot, chg, cslot


def _h_body(te_ref, lv_ref, fte_ref, fn_ref, fsl_ref, chg_ref, csl_ref,
            x_ref, g_hbm, hp_ref, h_ref, wbuf, sem):
    del hp_ref
    n = pl.program_id(0)
    i = pl.program_id(1)
    s = n * NT2 + i

    ft = fte_ref[s]

    @pl.when(ft >= 0)
    def _():
        fn = fn_ref[s]
        fs = fsl_ref[s]
        pltpu.make_async_copy(
            g_hbm.at[ft, :, pl.ds(fn * NB, NB)],
            wbuf.at[fs, 0], sem.at[fs]).start()
        pltpu.make_async_copy(
            g_hbm.at[ft, :, pl.ds(DI + fn * NB, NB)],
            wbuf.at[fs, 1], sem.at[fs]).start()

    cs = csl_ref[s]

    @pl.when(chg_ref[s] > 0)
    def _():
        te = te_ref[i]
        pltpu.make_async_copy(
            g_hbm.at[te, :, pl.ds(n * NB, NB)],
            wbuf.at[cs, 0], sem.at[cs]).wait()
        pltpu.make_async_copy(
            g_hbm.at[te, :, pl.ds(DI + n * NB, NB)],
            wbuf.at[cs, 1], sem.at[cs]).wait()

    @pl.when(lv_ref[i] > 0)
    def _():
        xb = x_ref[...]
        g = jnp.dot(xb, wbuf[cs, 0], preferred_element_type=jnp.float32)
        u = jnp.dot(xb, wbuf[cs, 1], preferred_element_type=jnp.float32)
        h = jax.nn.gelu(g, approximate=True) * u
        h_ref[...] = h.astype(jnp.bfloat16)


def _h_body_first(te_ref, lv_ref, fte_ref, fn_ref, fsl_ref, chg_ref,
                  csl_ref, x_ref, g_hbm, h_ref, wbuf, sem):
    _h_body(te_ref, lv_ref, fte_ref, fn_ref, fsl_ref, chg_ref, csl_ref,
            x_ref, g_hbm, None, h_ref, wbuf, sem)


def _tc_h(x_half, gate_up_proj, te_half, lv_half, half, h_prev=None):
    base = half * NT2
    f_te, f_n, f_slot, chg, cslot = _h_sched(te_half)
    in_specs = [
        pl.BlockSpec((TM, D), lambda n, i, *pref: (i, 0)),
        pl.BlockSpec(memory_space=pltpu.MemorySpace.HBM),
    ]
    args = [te_half, lv_half, f_te, f_n, f_slot, chg, cslot,
            x_half, gate_up_proj]
    if h_prev is None:
        body, aliases = _h_body_first, {}
    else:
        body, aliases = _h_body, {9: 0}
        in_specs = in_specs + [
            pl.BlockSpec(memory_space=pltpu.MemorySpace.HBM)]
        args = args + [h_prev]
    grid_spec = pltpu.PrefetchScalarGridSpec(
        num_scalar_prefetch=7,
        grid=(NN, NT2),
        in_specs=in_specs,
        out_specs=pl.BlockSpec((TM, NB), lambda n, i, *pref: (base + i, n)),
        scratch_shapes=[
            pltpu.VMEM((2, 2, D, NB), jnp.float32),
            pltpu.SemaphoreType.DMA((2,)),
        ],
    )
    return pl.pallas_call(
        body,
        grid_spec=grid_spec,
        out_shape=jax.ShapeDtypeStruct((NP, DI), jnp.bfloat16),
        input_output_aliases=aliases,
        compiler_params=pltpu.CompilerParams(
            dimension_semantics=("arbitrary", "arbitrary")),
    )(*args)


DM = D // 2


def _down_body(te_ref, lv_ref, h_ref, d_ref, w_ref, out_ref):
    i = pl.program_id(1)

    @pl.when(lv_ref[i] > 0)
    def _():
        hb = h_ref[...]
        dw = d_ref[0].astype(jnp.bfloat16)
        p = jnp.dot(hb, dw, preferred_element_type=jnp.float32)
        out_ref[...] = p * w_ref[0, 0, :][:, None]


def _tc_down(h_g, down_proj, slot_weight, tile_expert, tile_live):
    w3 = slot_weight.reshape(NT, 1, TM)
    grid_spec = pltpu.PrefetchScalarGridSpec(
        num_scalar_prefetch=2,
        grid=(2, NT),
        in_specs=[
            pl.BlockSpec((TM, DI), lambda m, i, te, lv: (i, 0)),
            pl.BlockSpec((1, DI, DM), lambda m, i, te, lv: (te[i], 0, m)),
            pl.BlockSpec((1, 1, TM), lambda m, i, te, lv: (i, 0, 0)),
        ],
        out_specs=pl.BlockSpec((TM, DM), lambda m, i, te, lv: (i, m)),
    )
    return pl.pallas_call(
        _down_body,
        grid_spec=grid_spec,
        out_shape=jax.ShapeDtypeStruct((NP, D), jnp.float32),
        compiler_params=pltpu.CompilerParams(
            dimension_semantics=("arbitrary", "arbitrary")),
    )(tile_expert, tile_live, h_g, down_proj, w3)


_C_CH = 8
_C_TOK = T // NW
_C_NCH = _C_TOK // _C_CH
_VR = D // 16


def _combine_body(hg_hbm, p0_hbm, p1_hbm, out_hbm, i0_v, i1_v,
                  a0, b0, a1, b1, ga0, gb0, ga1, gb1, w0, w1):
    wid = lax.axis_index("s") * NC + lax.axis_index("c")
    base = wid * _C_TOK
    pltpu.sync_copy(p0_hbm.at[pl.ds(base, _C_TOK)], i0_v)
    pltpu.sync_copy(p1_hbm.at[pl.ds(base, _C_TOK)], i1_v)

    bufs = ((a0, b0, ga0, gb0, w0), (a1, b1, ga1, gb1, w1))

    def g_start(j, p):
        a, b, ga, gb, _ = bufs[p]
        sl = pl.ds(j * _C_CH, _C_CH)
        pltpu.make_async_copy(hg_hbm.at[i0_v.at[sl]], a, ga).start()
        pltpu.make_async_copy(hg_hbm.at[i1_v.at[sl]], b, gb).start()

    def g_wait(p):
        a, b, ga, gb, _ = bufs[p]
        sl = pl.ds(0, _C_CH)
        pltpu.make_async_copy(hg_hbm.at[i0_v.at[sl]], a, ga).wait()
        pltpu.make_async_copy(hg_hbm.at[i1_v.at[sl]], b, gb).wait()

    def add_rows(p):
        a, b, _, _, _ = bufs[p]

        def row(r, carry2):
            def vec(j, carry3):
                sl = pl.ds(j * 16, 16)
                a[r, sl] = a[r, sl] + b[r, sl]
                return carry3
            return lax.fori_loop(0, _VR, vec, carry2, unroll=8)

        lax.fori_loop(0, _C_CH, row, 0)

    def w_start(j, p):
        a, _, _, _, w = bufs[p]
        pltpu.make_async_copy(
            a, out_hbm.at[pl.ds(base + j * _C_CH, _C_CH)], w).start()

    def w_wait(p):
        a, _, _, _, w = bufs[p]
        pltpu.make_async_copy(a, out_hbm.at[pl.ds(base, _C_CH)], w).wait()

    g_start(0, 0)
    g_start(1, 1)

    def chunk(k, carry):
        for p in range(2):
            j = 2 * k + p
            g_wait(p)
            add_rows(p)
            w_start(j, p)
        for p in range(2):
            jn = 2 * (k + 1) + p
            w_wait(p)

            @pl.when(jn < _C_NCH)
            def _():
                g_start(jn, p)
        return carry

    lax.fori_loop(0, _C_NCH // 2, chunk, 0)


def _sc_combine(h_g, inv_pos):
    p0 = inv_pos[:, 0]
    p1 = inv_pos[:, 1]
    return pl.kernel(
        _combine_body,
        out_type=jax.ShapeDtypeStruct((T, D), jnp.float32),
        mesh=plsc.VectorSubcoreMesh(**_SC_MESH),
        scratch_types=(
            [pltpu.VMEM((_C_TOK,), jnp.int32)] * 2
            + [pltpu.VMEM((_C_CH, D), jnp.float32) for _ in range(4)]
            + [pltpu.SemaphoreType.DMA] * 6
        ),
    )(h_g, p0, p1)


def kernel(x, top_k_index, top_k_weights, gate_up_proj, down_proj):
    slot_token, slot_weight, inv_pos, tile_expert, tile_live = _routing(
        top_k_index, top_k_weights)
    half = NP // 2
    x_g0 = _sc_gather(x, slot_token[:half])
    x_g1 = _sc_gather(x, slot_token[half:])
    h_g = _tc_h(x_g0, gate_up_proj, tile_expert[:NT2], tile_live[:NT2], 0)
    h_g = _tc_h(x_g1, gate_up_proj, tile_expert[NT2:], tile_live[NT2:],
                1, h_g)
    out_g = _tc_down(h_g, down_proj, slot_weight, tile_expert, tile_live)
    return _sc_combine(out_g, inv_pos)


# speedup vs baseline: 1.7555x; 1.0067x over previous
import functools

import jax
import jax.numpy as jnp
from jax import lax
from jax.experimental import pallas as pl
from jax.experimental.pallas import tpu as pltpu
from jax.experimental.pallas import tpu_sc as plsc

E = 8
D = 2048
DI = 4096
T = 4096
K = 2
S = T * K

TM = 256
NP = S + E * TM
NT = NP // TM
NB = 1024
NN = DI // NB

NC, NS = 2, 16
NW = NC * NS

_SC_MESH = dict(core_axis_name="c", subcore_axis_name="s",
                num_cores=NC, num_subcores=NS)


def _routing(top_k_index, top_k_weights):
    expert = top_k_index.reshape(-1).astype(jnp.int32)
    token = jnp.arange(S, dtype=jnp.int32) // K
    occ = (expert[:, None] == jnp.arange(E, dtype=jnp.int32)[None, :]
           ).astype(jnp.int32)
    ranks = jnp.cumsum(occ, axis=0) - occ
    counts = jnp.sum(occ, axis=0)
    padded = ((counts + TM - 1) // TM) * TM
    padded_off = jnp.concatenate(
        [jnp.zeros(1, jnp.int32), jnp.cumsum(padded)]).astype(jnp.int32)
    rank = jnp.take_along_axis(ranks, expert[:, None], axis=1)[:, 0]
    pos = padded_off[expert] + rank
    slot_token = jnp.zeros(NP, jnp.int32).at[pos].set(token)
    slot_weight = jnp.zeros(NP, jnp.float32).at[pos].set(
        top_k_weights.reshape(-1))
    inv_pos = pos.reshape(T, K)
    tile_expert = jnp.searchsorted(
        padded_off, jnp.arange(NT, dtype=jnp.int32) * TM,
        side="right").astype(jnp.int32) - 1
    tile_expert = jnp.clip(tile_expert, 0, E - 1)
    tile_live = (jnp.arange(NT, dtype=jnp.int32) * TM
                 < padded_off[E]).astype(jnp.int32)
    return slot_token, slot_weight, inv_pos, tile_expert, tile_live


_G_CH = 16
_G_ROWS = (NP // 2) // NW
_G_NCH = _G_ROWS // _G_CH
_G_NBUF = 2


def _gather_body(x_hbm, idx_hbm, out_hbm, idx_v, *bufs_sems):
    bufs = bufs_sems[:_G_NBUF]
    gsem = bufs_sems[_G_NBUF:2 * _G_NBUF]
    wsem = bufs_sems[2 * _G_NBUF:3 * _G_NBUF]
    wid = lax.axis_index("s") * NC + lax.axis_index("c")
    base = wid * _G_ROWS
    pltpu.sync_copy(idx_hbm.at[pl.ds(base, _G_ROWS)], idx_v)

    def g_start(j, b):
        pltpu.make_async_copy(
            x_hbm.at[idx_v.at[pl.ds(j * _G_CH, _G_CH)]],
            bufs[b], gsem[b]).start()

    def g_wait(b):
        pltpu.make_async_copy(
            x_hbm.at[idx_v.at[pl.ds(0, _G_CH)]], bufs[b], gsem[b]).wait()

    def w_start(j, b):
        pltpu.make_async_copy(
            bufs[b], out_hbm.at[pl.ds(base + j * _G_CH, _G_CH)],
            wsem[b]).start()

    def w_wait(b):
        pltpu.make_async_copy(
            bufs[b], out_hbm.at[pl.ds(base, _G_CH)], wsem[b]).wait()

    for b in range(_G_NBUF):
        g_start(b, b)

    def ring(k, carry):
        for b in range(_G_NBUF):
            j = _G_NBUF * k + b
            g_wait(b)
            w_start(j, b)
        for b in range(_G_NBUF):
            jn = _G_NBUF * (k + 1) + b
            w_wait(b)

            @pl.when(jn < _G_NCH)
            def _():
                g_start(jn, b)
        return carry

    lax.fori_loop(0, _G_NCH // _G_NBUF, ring, 0)


def _sc_gather(x, slot_token_half):
    return pl.kernel(
        _gather_body,
        out_type=jax.ShapeDtypeStruct((NP // 2, D), jnp.float32),
        mesh=plsc.VectorSubcoreMesh(**_SC_MESH),
        scratch_types=(
            [pltpu.VMEM((_G_ROWS,), jnp.int32)]
            + [pltpu.VMEM((_G_CH, D), jnp.float32) for _ in range(_G_NBUF)]
            + [pltpu.SemaphoreType.DMA] * (2 * _G_NBUF)
        ),
    )(x, slot_token_half)


NT2 = NT // 2


_LA = 3


def _mk_sched(te_s, n_s):
    _S1 = te_s.shape[0]
    s = jnp.arange(_S1, dtype=jnp.int32)
    prev_te = jnp.concatenate([jnp.full((1,), -1, jnp.int32), te_s[:-1]])
    prev_n = jnp.concatenate([jnp.full((1,), -1, jnp.int32), n_s[:-1]])
    chg = ((te_s != prev_te) | (n_s != prev_n)).astype(jnp.int32)
    blk = jnp.cumsum(chg) - 1
    nblk = blk[-1] + 1
    big = jnp.int32(10 * _S1)
    fu = jnp.full((_S1,), big, jnp.int32).at[blk].min(s)
    kk = jnp.arange(_S1, dtype=jnp.int32)
    fu_prev = jnp.concatenate([jnp.full((1,), -big, jnp.int32), fu[:-1]])
    issue = jnp.maximum(fu - _LA, fu_prev)
    issue = kk + jax.lax.cummax(issue - kk, axis=0)
    issue = jnp.maximum(issue, kk)
    issue = jnp.where(kk < nblk, issue, big)
    fusafe = jnp.clip(fu, 0, _S1 - 1)
    f_te = jnp.full((_S1,), -1, jnp.int32).at[issue].set(
        te_s[fusafe], mode="drop")
    f_n = jnp.zeros((_S1,), jnp.int32).at[issue].set(
        n_s[fusafe], mode="drop")
    f_slot = jnp.zeros((_S1,), jnp.int32).at[issue].set(
        kk % 2, mode="drop")
    cslot = blk % 2
    return f_te, f_n, f_slot, chg, cslot


def _h_body(te_ref, lv_ref, fte_ref, fn_ref, fsl_ref, chg_ref, csl_ref,
            x_ref, g_hbm, hp_ref, h_ref, wbuf, sem):
    del hp_ref
    n = pl.program_id(0)
    i = pl.program_id(1)
    s = n * NT2 + i

    ft = fte_ref[s]

    @pl.when(ft >= 0)
    def _():
        fn = fn_ref[s]
        fs = fsl_ref[s]
        pltpu.make_async_copy(
            g_hbm.at[ft, :, pl.ds(fn * NB, NB)],
            wbuf.at[fs, 0], sem.at[fs]).start()
        pltpu.make_async_copy(
            g_hbm.at[ft, :, pl.ds(DI + fn * NB, NB)],
            wbuf.at[fs, 1], sem.at[fs]).start()

    cs = csl_ref[s]

    @pl.when(chg_ref[s] > 0)
    def _():
        te = te_ref[i]
        pltpu.make_async_copy(
            g_hbm.at[te, :, pl.ds(n * NB, NB)],
            wbuf.at[cs, 0], sem.at[cs]).wait()
        pltpu.make_async_copy(
            g_hbm.at[te, :, pl.ds(DI + n * NB, NB)],
            wbuf.at[cs, 1], sem.at[cs]).wait()

    @pl.when(lv_ref[i] > 0)
    def _():
        xb = x_ref[...]
        g = jnp.dot(xb, wbuf[cs, 0], preferred_element_type=jnp.float32)
        u = jnp.dot(xb, wbuf[cs, 1], preferred_element_type=jnp.float32)
        h = jax.nn.gelu(g, approximate=True) * u
        h_ref[...] = h.astype(jnp.bfloat16)


def _h_body_first(te_ref, lv_ref, fte_ref, fn_ref, fsl_ref, chg_ref,
                  csl_ref, x_ref, g_hbm, h_ref, wbuf, sem):
    _h_body(te_ref, lv_ref, fte_ref, fn_ref, fsl_ref, chg_ref, csl_ref,
            x_ref, g_hbm, None, h_ref, wbuf, sem)


def _tc_h(x_half, gate_up_proj, te_half, lv_half, half, h_prev=None):
    base = half * NT2
    s = jnp.arange(NN * NT2, dtype=jnp.int32)
    f_te, f_n, f_slot, chg, cslot = _mk_sched(te_half[s % NT2], s // NT2)
    in_specs = [
        pl.BlockSpec((TM, D), lambda n, i, *pref: (i, 0)),
        pl.BlockSpec(memory_space=pltpu.MemorySpace.HBM),
    ]
    args = [te_half, lv_half, f_te, f_n, f_slot, chg, cslot,
            x_half, gate_up_proj]
    if h_prev is None:
        body, aliases = _h_body_first, {}
    else:
        body, aliases = _h_body, {9: 0}
        in_specs = in_specs + [
            pl.BlockSpec(memory_space=pltpu.MemorySpace.HBM)]
        args = args + [h_prev]
    grid_spec = pltpu.PrefetchScalarGridSpec(
        num_scalar_prefetch=7,
        grid=(NN, NT2),
        in_specs=in_specs,
        out_specs=pl.BlockSpec((TM, NB), lambda n, i, *pref: (base + i, n)),
        scratch_shapes=[
            pltpu.VMEM((2, 2, D, NB), jnp.float32),
            pltpu.SemaphoreType.DMA((2,)),
        ],
    )
    return pl.pallas_call(
        body,
        grid_spec=grid_spec,
        out_shape=jax.ShapeDtypeStruct((NP, DI), jnp.bfloat16),
        input_output_aliases=aliases,
        compiler_params=pltpu.CompilerParams(
            dimension_semantics=("arbitrary", "arbitrary")),
    )(*args)


DM = D // 2


def _down_body(te_ref, lv_ref, fte_ref, fm_ref, fsl_ref, chg_ref, csl_ref,
               h_ref, d_hbm, w_ref, out_ref, wbuf, sem):
    m = pl.program_id(0)
    i = pl.program_id(1)
    s = m * NT + i

    ft = fte_ref[s]

    @pl.when(ft >= 0)
    def _():
        fm = fm_ref[s]
        fs = fsl_ref[s]
        pltpu.make_async_copy(
            d_hbm.at[ft, :, pl.ds(fm * DM, DM)],
            wbuf.at[fs], sem.at[fs]).start()

    cs = csl_ref[s]

    @pl.when(chg_ref[s] > 0)
    def _():
        pltpu.make_async_copy(
            d_hbm.at[te_ref[i], :, pl.ds(m * DM, DM)],
            wbuf.at[cs], sem.at[cs]).wait()

    @pl.when(lv_ref[i] > 0)
    def _():
        hb = h_ref[...]
        dw = wbuf[cs].astype(jnp.bfloat16)
        p = jnp.dot(hb, dw, preferred_element_type=jnp.float32)
        out_ref[...] = p * w_ref[0, 0, :][:, None]


def _tc_down(h_g, down_proj, slot_weight, tile_expert, tile_live):
    w3 = slot_weight.reshape(NT, 1, TM)
    s = jnp.arange(2 * NT, dtype=jnp.int32)
    f_te, f_m, f_slot, chg, cslot = _mk_sched(tile_expert[s % NT], s // NT)
    grid_spec = pltpu.PrefetchScalarGridSpec(
        num_scalar_prefetch=7,
        grid=(2, NT),
        in_specs=[
            pl.BlockSpec((TM, DI), lambda m, i, *pref: (i, 0)),
            pl.BlockSpec(memory_space=pltpu.MemorySpace.HBM),
            pl.BlockSpec((1, 1, TM), lambda m, i, *pref: (i, 0, 0)),
        ],
        out_specs=pl.BlockSpec((TM, DM), lambda m, i, *pref: (i, m)),
        scratch_shapes=[
            pltpu.VMEM((2, DI, DM), jnp.float32),
            pltpu.SemaphoreType.DMA((2,)),
        ],
    )
    return pl.pallas_call(
        _down_body,
        grid_spec=grid_spec,
        out_shape=jax.ShapeDtypeStruct((NP, D), jnp.float32),
        compiler_params=pltpu.CompilerParams(
            dimension_semantics=("arbitrary", "arbitrary")),
    )(tile_expert, tile_live, f_te, f_m, f_slot, chg, cslot,
      h_g, down_proj, w3)


_C_CH = 8
_C_TOK = T // NW
_C_NCH = _C_TOK // _C_CH
_VR = D // 16


def _combine_body(hg_hbm, p0_hbm, p1_hbm, out_hbm, i0_v, i1_v,
                  a0, b0, a1, b1, ga0, gb0, ga1, gb1, w0, w1):
    wid = lax.axis_index("s") * NC + lax.axis_index("c")
    base = wid * _C_TOK
    pltpu.sync_copy(p0_hbm.at[pl.ds(base, _C_TOK)], i0_v)
    pltpu.sync_copy(p1_hbm.at[pl.ds(base, _C_TOK)], i1_v)

    bufs = ((a0, b0, ga0, gb0, w0), (a1, b1, ga1, gb1, w1))

    def g_start(j, p):
        a, b, ga, gb, _ = bufs[p]
        sl = pl.ds(j * _C_CH, _C_CH)
        pltpu.make_async_copy(hg_hbm.at[i0_v.at[sl]], a, ga).start()
        pltpu.make_async_copy(hg_hbm.at[i1_v.at[sl]], b, gb).start()

    def g_wait(p):
        a, b, ga, gb, _ = bufs[p]
        sl = pl.ds(0, _C_CH)
        pltpu.make_async_copy(hg_hbm.at[i0_v.at[sl]], a, ga).wait()
        pltpu.make_async_copy(hg_hbm.at[i1_v.at[sl]], b, gb).wait()

    def add_rows(p):
        a, b, _, _, _ = bufs[p]

        def row(r, carry2):
            def vec(j, carry3):
                sl = pl.ds(j * 16, 16)
                a[r, sl] = a[r, sl] + b[r, sl]
                return carry3
            return lax.fori_loop(0, _VR, vec, carry2, unroll=8)

        lax.fori_loop(0, _C_CH, row, 0)

    def w_start(j, p):
        a, _, _, _, w = bufs[p]
        pltpu.make_async_copy(
            a, out_hbm.at[pl.ds(base + j * _C_CH, _C_CH)], w).start()

    def w_wait(p):
        a, _, _, _, w = bufs[p]
        pltpu.make_async_copy(a, out_hbm.at[pl.ds(base, _C_CH)], w).wait()

    g_start(0, 0)
    g_start(1, 1)

    def chunk(k, carry):
        for p in range(2):
            j = 2 * k + p
            g_wait(p)
            add_rows(p)
            w_start(j, p)
        for p in range(2):
            jn = 2 * (k + 1) + p
            w_wait(p)

            @pl.when(jn < _C_NCH)
            def _():
                g_start(jn, p)
        return carry

    lax.fori_loop(0, _C_NCH // 2, chunk, 0)


def _sc_combine(h_g, inv_pos):
    p0 = inv_pos[:, 0]
    p1 = inv_pos[:, 1]
    return pl.kernel(
        _combine_body,
        out_type=jax.ShapeDtypeStruct((T, D), jnp.float32),
        mesh=plsc.VectorSubcoreMesh(**_SC_MESH),
        scratch_types=(
            [pltpu.VMEM((_C_TOK,), jnp.int32)] * 2
            + [pltpu.VMEM((_C_CH, D), jnp.float32) for _ in range(4)]
            + [pltpu.SemaphoreType.DMA] * 6
        ),
    )(h_g, p0, p1)


def kernel(x, top_k_index, top_k_weights, gate_up_proj, down_proj):
    slot_token, slot_weight, inv_pos, tile_expert, tile_live = _routing(
        top_k_index, top_k_weights)
    half = NP // 2
    x_g0 = _sc_gather(x, slot_token[:half])
    x_g1 = _sc_gather(x, slot_token[half:])
    h_g = _tc_h(x_g0, gate_up_proj, tile_expert[:NT2], tile_live[:NT2], 0)
    h_g = _tc_h(x_g1, gate_up_proj, tile_expert[NT2:], tile_live[NT2:],
                1, h_g)
    out_g = _tc_down(h_g, down_proj, slot_weight, tile_expert, tile_live)
    return _sc_combine(out_g, inv_pos)


# speedup vs baseline: 1.7593x; 1.0022x over previous
import functools

import jax
import jax.numpy as jnp
from jax import lax
from jax.experimental import pallas as pl
from jax.experimental.pallas import tpu as pltpu
from jax.experimental.pallas import tpu_sc as plsc

E = 8
D = 2048
DI = 4096
T = 4096
K = 2
S = T * K

TM = 256
NP = S + E * TM
NT = NP // TM
NB = 1024
NN = DI // NB

NC, NS = 2, 16
NW = NC * NS

_SC_MESH = dict(core_axis_name="c", subcore_axis_name="s",
                num_cores=NC, num_subcores=NS)


def _routing(top_k_index, top_k_weights):
    expert = top_k_index.reshape(-1).astype(jnp.int32)
    token = jnp.arange(S, dtype=jnp.int32) // K
    occ = (expert[:, None] == jnp.arange(E, dtype=jnp.int32)[None, :]
           ).astype(jnp.int32)
    ranks = jnp.cumsum(occ, axis=0) - occ
    counts = jnp.sum(occ, axis=0)
    padded = ((counts + TM - 1) // TM) * TM
    padded_off = jnp.concatenate(
        [jnp.zeros(1, jnp.int32), jnp.cumsum(padded)]).astype(jnp.int32)
    rank = jnp.take_along_axis(ranks, expert[:, None], axis=1)[:, 0]
    pos = padded_off[expert] + rank
    slot_token = jnp.zeros(NP, jnp.int32).at[pos].set(token)
    slot_weight = jnp.zeros(NP, jnp.float32).at[pos].set(
        top_k_weights.reshape(-1))
    inv_pos = pos.reshape(T, K)
    tile_expert = jnp.searchsorted(
        padded_off, jnp.arange(NT, dtype=jnp.int32) * TM,
        side="right").astype(jnp.int32) - 1
    tile_expert = jnp.clip(tile_expert, 0, E - 1)
    tile_live = (jnp.arange(NT, dtype=jnp.int32) * TM
                 < padded_off[E]).astype(jnp.int32)
    return slot_token, slot_weight, inv_pos, tile_expert, tile_live


_G_CH = 16
_G_ROWS = (NP // 2) // NW
_G_NCH = _G_ROWS // _G_CH
_G_NBUF = 2


def _gather_body(x_hbm, idx_hbm, out_hbm, idx_v, *bufs_sems):
    bufs = bufs_sems[:_G_NBUF]
    gsem = bufs_sems[_G_NBUF:2 * _G_NBUF]
    wsem = bufs_sems[2 * _G_NBUF:3 * _G_NBUF]
    wid = lax.axis_index("s") * NC + lax.axis_index("c")
    base = wid * _G_ROWS
    pltpu.sync_copy(idx_hbm.at[pl.ds(base, _G_ROWS)], idx_v)

    def g_start(j, b):
        pltpu.make_async_copy(
            x_hbm.at[idx_v.at[pl.ds(j * _G_CH, _G_CH)]],
            bufs[b], gsem[b]).start()

    def g_wait(b):
        pltpu.make_async_copy(
            x_hbm.at[idx_v.at[pl.ds(0, _G_CH)]], bufs[b], gsem[b]).wait()

    def w_start(j, b):
        pltpu.make_async_copy(
            bufs[b], out_hbm.at[pl.ds(base + j * _G_CH, _G_CH)],
            wsem[b]).start()

    def w_wait(b):
        pltpu.make_async_copy(
            bufs[b], out_hbm.at[pl.ds(base, _G_CH)], wsem[b]).wait()

    for b in range(_G_NBUF):
        g_start(b, b)

    def ring(k, carry):
        for b in range(_G_NBUF):
            j = _G_NBUF * k + b
            g_wait(b)
            w_start(j, b)
        for b in range(_G_NBUF):
            jn = _G_NBUF * (k + 1) + b
            w_wait(b)

            @pl.when(jn < _G_NCH)
            def _():
                g_start(jn, b)
        return carry

    lax.fori_loop(0, _G_NCH // _G_NBUF, ring, 0)


def _sc_gather(x, slot_token_half):
    return pl.kernel(
        _gather_body,
        out_type=jax.ShapeDtypeStruct((NP // 2, D), jnp.float32),
        mesh=plsc.VectorSubcoreMesh(**_SC_MESH),
        scratch_types=(
            [pltpu.VMEM((_G_ROWS,), jnp.int32)]
            + [pltpu.VMEM((_G_CH, D), jnp.float32) for _ in range(_G_NBUF)]
            + [pltpu.SemaphoreType.DMA] * (2 * _G_NBUF)
        ),
    )(x, slot_token_half)


NT2 = NT // 2


_LA = 5


def _mk_sched(te_s, n_s):
    _S1 = te_s.shape[0]
    s = jnp.arange(_S1, dtype=jnp.int32)
    prev_te = jnp.concatenate([jnp.full((1,), -1, jnp.int32), te_s[:-1]])
    prev_n = jnp.concatenate([jnp.full((1,), -1, jnp.int32), n_s[:-1]])
    chg = ((te_s != prev_te) | (n_s != prev_n)).astype(jnp.int32)
    blk = jnp.cumsum(chg) - 1
    nblk = blk[-1] + 1
    big = jnp.int32(10 * _S1)
    fu = jnp.full((_S1,), big, jnp.int32).at[blk].min(s)
    kk = jnp.arange(_S1, dtype=jnp.int32)
    fu_prev = jnp.concatenate([jnp.full((1,), -big, jnp.int32), fu[:-1]])
    issue = jnp.maximum(fu - _LA, fu_prev)
    issue = kk + jax.lax.cummax(issue - kk, axis=0)
    issue = jnp.maximum(issue, kk)
    issue = jnp.where(kk < nblk, issue, big)
    fusafe = jnp.clip(fu, 0, _S1 - 1)
    f_te = jnp.full((_S1,), -1, jnp.int32).at[issue].set(
        te_s[fusafe], mode="drop")
    f_n = jnp.zeros((_S1,), jnp.int32).at[issue].set(
        n_s[fusafe], mode="drop")
    f_slot = jnp.zeros((_S1,), jnp.int32).at[issue].set(
        kk % 2, mode="drop")
    cslot = blk % 2
    return f_te, f_n, f_slot, chg, cslot


def _h_body(te_ref, lv_ref, fte_ref, fn_ref, fsl_ref, chg_ref, csl_ref,
            x_ref, g_hbm, hp_ref, h_ref, wbuf, sem):
    del hp_ref
    n = pl.program_id(0)
    i = pl.program_id(1)
    s = n * NT2 + i

    ft = fte_ref[s]

    @pl.when(ft >= 0)
    def _():
        fn = fn_ref[s]
        fs = fsl_ref[s]
        pltpu.make_async_copy(
            g_hbm.at[ft, :, pl.ds(fn * NB, NB)],
            wbuf.at[fs, 0], sem.at[fs]).start()
        pltpu.make_async_copy(
            g_hbm.at[ft, :, pl.ds(DI + fn * NB, NB)],
            wbuf.at[fs, 1], sem.at[fs]).start()

    cs = csl_ref[s]

    @pl.when(chg_ref[s] > 0)
    def _():
        te = te_ref[i]
        pltpu.make_async_copy(
            g_hbm.at[te, :, pl.ds(n * NB, NB)],
            wbuf.at[cs, 0], sem.at[cs]).wait()
        pltpu.make_async_copy(
            g_hbm.at[te, :, pl.ds(DI + n * NB, NB)],
            wbuf.at[cs, 1], sem.at[cs]).wait()

    @pl.when(lv_ref[i] > 0)
    def _():
        xb = x_ref[...]
        g = jnp.dot(xb, wbuf[cs, 0], preferred_element_type=jnp.float32)
        u = jnp.dot(xb, wbuf[cs, 1], preferred_element_type=jnp.float32)
        h = jax.nn.gelu(g, approximate=True) * u
        h_ref[...] = h.astype(jnp.bfloat16)


def _h_body_first(te_ref, lv_ref, fte_ref, fn_ref, fsl_ref, chg_ref,
                  csl_ref, x_ref, g_hbm, h_ref, wbuf, sem):
    _h_body(te_ref, lv_ref, fte_ref, fn_ref, fsl_ref, chg_ref, csl_ref,
            x_ref, g_hbm, None, h_ref, wbuf, sem)


def _tc_h(x_half, gate_up_proj, te_half, lv_half, half, h_prev=None):
    base = half * NT2
    s = jnp.arange(NN * NT2, dtype=jnp.int32)
    f_te, f_n, f_slot, chg, cslot = _mk_sched(te_half[s % NT2], s // NT2)
    in_specs = [
        pl.BlockSpec((TM, D), lambda n, i, *pref: (i, 0)),
        pl.BlockSpec(memory_space=pltpu.MemorySpace.HBM),
    ]
    args = [te_half, lv_half, f_te, f_n, f_slot, chg, cslot,
            x_half, gate_up_proj]
    if h_prev is None:
        body, aliases = _h_body_first, {}
    else:
        body, aliases = _h_body, {9: 0}
        in_specs = in_specs + [
            pl.BlockSpec(memory_space=pltpu.MemorySpace.HBM)]
        args = args + [h_prev]
    grid_spec = pltpu.PrefetchScalarGridSpec(
        num_scalar_prefetch=7,
        grid=(NN, NT2),
        in_specs=in_specs,
        out_specs=pl.BlockSpec((TM, NB), lambda n, i, *pref: (base + i, n)),
        scratch_shapes=[
            pltpu.VMEM((2, 2, D, NB), jnp.float32),
            pltpu.SemaphoreType.DMA((2,)),
        ],
    )
    return pl.pallas_call(
        body,
        grid_spec=grid_spec,
        out_shape=jax.ShapeDtypeStruct((NP, DI), jnp.bfloat16),
        input_output_aliases=aliases,
        compiler_params=pltpu.CompilerParams(
            dimension_semantics=("arbitrary", "arbitrary")),
    )(*args)


DM = D // 2


def _down_body(te_ref, lv_ref, fte_ref, fm_ref, fsl_ref, chg_ref, csl_ref,
               h_ref, d_hbm, w_ref, out_ref, wbuf, sem):
    m = pl.program_id(0)
    i = pl.program_id(1)
    s = m * NT + i

    ft = fte_ref[s]

    @pl.when(ft >= 0)
    def _():
        fm = fm_ref[s]
        fs = fsl_ref[s]
        pltpu.make_async_copy(
            d_hbm.at[ft, :, pl.ds(fm * DM, DM)],
            wbuf.at[fs], sem.at[fs]).start()

    cs = csl_ref[s]

    @pl.when(chg_ref[s] > 0)
    def _():
        pltpu.make_async_copy(
            d_hbm.at[te_ref[i], :, pl.ds(m * DM, DM)],
            wbuf.at[cs], sem.at[cs]).wait()

    @pl.when(lv_ref[i] > 0)
    def _():
        hb = h_ref[...]
        dw = wbuf[cs].astype(jnp.bfloat16)
        p = jnp.dot(hb, dw, preferred_element_type=jnp.float32)
        out_ref[...] = p * w_ref[0, 0, :][:, None]


def _tc_down(h_g, down_proj, slot_weight, tile_expert, tile_live):
    w3 = slot_weight.reshape(NT, 1, TM)
    s = jnp.arange(2 * NT, dtype=jnp.int32)
    f_te, f_m, f_slot, chg, cslot = _mk_sched(tile_expert[s % NT], s // NT)
    grid_spec = pltpu.PrefetchScalarGridSpec(
        num_scalar_prefetch=7,
        grid=(2, NT),
        in_specs=[
            pl.BlockSpec((TM, DI), lambda m, i, *pref: (i, 0)),
            pl.BlockSpec(memory_space=pltpu.MemorySpace.HBM),
            pl.BlockSpec((1, 1, TM), lambda m, i, *pref: (i, 0, 0)),
        ],
        out_specs=pl.BlockSpec((TM, DM), lambda m, i, *pref: (i, m)),
        scratch_shapes=[
            pltpu.VMEM((2, DI, DM), jnp.float32),
            pltpu.SemaphoreType.DMA((2,)),
        ],
    )
    return pl.pallas_call(
        _down_body,
        grid_spec=grid_spec,
        out_shape=jax.ShapeDtypeStruct((NP, D), jnp.float32),
        compiler_params=pltpu.CompilerParams(
            dimension_semantics=("arbitrary", "arbitrary")),
    )(tile_expert, tile_live, f_te, f_m, f_slot, chg, cslot,
      h_g, down_proj, w3)


_C_CH = 8
_C_TOK = T // NW
_C_NCH = _C_TOK // _C_CH
_VR = D // 16


def _combine_body(hg_hbm, p0_hbm, p1_hbm, out_hbm, i0_v, i1_v,
                  a0, b0, a1, b1, ga0, gb0, ga1, gb1, w0, w1):
    wid = lax.axis_index("s") * NC + lax.axis_index("c")
    base = wid * _C_TOK
    pltpu.sync_copy(p0_hbm.at[pl.ds(base, _C_TOK)], i0_v)
    pltpu.sync_copy(p1_hbm.at[pl.ds(base, _C_TOK)], i1_v)

    bufs = ((a0, b0, ga0, gb0, w0), (a1, b1, ga1, gb1, w1))

    def g_start(j, p):
        a, b, ga, gb, _ = bufs[p]
        sl = pl.ds(j * _C_CH, _C_CH)
        pltpu.make_async_copy(hg_hbm.at[i0_v.at[sl]], a, ga).start()
        pltpu.make_async_copy(hg_hbm.at[i1_v.at[sl]], b, gb).start()

    def g_wait(p):
        a, b, ga, gb, _ = bufs[p]
        sl = pl.ds(0, _C_CH)
        pltpu.make_async_copy(hg_hbm.at[i0_v.at[sl]], a, ga).wait()
        pltpu.make_async_copy(hg_hbm.at[i1_v.at[sl]], b, gb).wait()

    def add_rows(p):
        a, b, _, _, _ = bufs[p]

        def row(r, carry2):
            def vec(j, carry3):
                sl = pl.ds(j * 16, 16)
                a[r, sl] = a[r, sl] + b[r, sl]
                return carry3
            return lax.fori_loop(0, _VR, vec, carry2, unroll=8)

        lax.fori_loop(0, _C_CH, row, 0)

    def w_start(j, p):
        a, _, _, _, w = bufs[p]
        pltpu.make_async_copy(
            a, out_hbm.at[pl.ds(base + j * _C_CH, _C_CH)], w).start()

    def w_wait(p):
        a, _, _, _, w = bufs[p]
        pltpu.make_async_copy(a, out_hbm.at[pl.ds(base, _C_CH)], w).wait()

    g_start(0, 0)
    g_start(1, 1)

    def chunk(k, carry):
        for p in range(2):
            j = 2 * k + p
            g_wait(p)
            add_rows(p)
            w_start(j, p)
        for p in range(2):
            jn = 2 * (k + 1) + p
            w_wait(p)

            @pl.when(jn < _C_NCH)
            def _():
                g_start(jn, p)
        return carry

    lax.fori_loop(0, _C_NCH // 2, chunk, 0)


def _sc_combine(h_g, inv_pos):
    p0 = inv_pos[:, 0]
    p1 = inv_pos[:, 1]
    return pl.kernel(
        _combine_body,
        out_type=jax.ShapeDtypeStruct((T, D), jnp.float32),
        mesh=plsc.VectorSubcoreMesh(**_SC_MESH),
        scratch_types=(
            [pltpu.VMEM((_C_TOK,), jnp.int32)] * 2
            + [pltpu.VMEM((_C_CH, D), jnp.float32) for _ in range(4)]
            + [pltpu.SemaphoreType.DMA] * 6
        ),
    )(h_g, p0, p1)


def kernel(x, top_k_index, top_k_weights, gate_up_proj, down_proj):
    slot_token, slot_weight, inv_pos, tile_expert, tile_live = _routing(
        top_k_index, top_k_weights)
    half = NP // 2
    x_g0 = _sc_gather(x, slot_token[:half])
    x_g1 = _sc_gather(x, slot_token[half:])
    h_g = _tc_h(x_g0, gate_up_proj, tile_expert[:NT2], tile_live[:NT2], 0)
    h_g = _tc_h(x_g1, gate_up_proj, tile_expert[NT2:], tile_live[NT2:],
                1, h_g)
    out_g = _tc_down(h_g, down_proj, slot_weight, tile_expert, tile_live)
    return _sc_combine(out_g, inv_pos)
